# Initial kernel scaffold; baseline (speedup 1.0000x reference)
import jax
import jax.numpy as jnp
from jax.experimental import pallas as pl


def kernel(type, all_other_agents_types, lanes_mid, lanes, embedding_weight):
    raise NotImplementedError("write your pallas kernel here")


# speedup vs baseline: 7.8243x; 7.8243x over previous
import jax
import jax.numpy as jnp
from jax.experimental import pallas as pl

B = 1024
OTHER_LEN = 50
LANES_LEN = 200
BDRY_LEN = 200
EMB_DIM = 64
TOTAL_LEN = 1 + OTHER_LEN + LANES_LEN + BDRY_LEN

_BB = 32


def _tc_body(aoat_ref, w_ref, out_ref):
    mask = aoat_ref[...] == 1
    w0 = w_ref[0:1, :, :]
    w1 = w_ref[1:2, :, :]
    w2 = w_ref[2:3, :, :]
    w5 = w_ref[5:6, :, :]
    w11 = w_ref[11:12, :, :]

    out_ref[:, 0:1, :] = jnp.broadcast_to(w0, (_BB, 1, EMB_DIM))
    out_ref[:, 1:1 + OTHER_LEN, :] = jnp.where(mask, w2, w1)
    out_ref[:, 1 + OTHER_LEN:1 + OTHER_LEN + LANES_LEN, :] = jnp.broadcast_to(
        w5, (_BB, LANES_LEN, EMB_DIM))
    out_ref[:, 1 + OTHER_LEN + LANES_LEN:, :] = jnp.broadcast_to(
        w11, (_BB, BDRY_LEN, EMB_DIM))


def kernel(type, all_other_agents_types, lanes_mid, lanes, embedding_weight):
    del type, lanes_mid, lanes
    aoat = all_other_agents_types.astype(jnp.int32).reshape(B, OTHER_LEN, 1)
    w3 = embedding_weight.reshape(12, 1, EMB_DIM)
    return pl.pallas_call(
        _tc_body,
        grid=(B // _BB,),
        in_specs=[
            pl.BlockSpec((_BB, OTHER_LEN, 1), lambda i: (i, 0, 0)),
            pl.BlockSpec((12, 1, EMB_DIM), lambda i: (0, 0, 0)),
        ],
        out_specs=pl.BlockSpec((_BB, TOTAL_LEN, EMB_DIM), lambda i: (i, 0, 0)),
        out_shape=jax.ShapeDtypeStruct((B, TOTAL_LEN, EMB_DIM), jnp.float32),
    )(aoat, w3)
